# baseline (speedup 1.0000x reference)
import jax
import jax.numpy as jnp
from jax import lax
from jax.experimental import pallas as pl
from jax.experimental.pallas import tpu as pltpu
from jax.experimental.pallas import tpu_sc as plsc

_N = 10000
_E = 320000
_D = 128
_C = 8
_NC = 2
_NS = 16
_NW = _NC * _NS
_EPW = _E // _NW
_CH = 2000
_NCHUNK = _EPW // _CH
_GRP = _CH // 16
_UNROLL = 5
_NPAD = 10240
_NGRP = _NPAD // 16
_GPT = _NGRP // _NS
_SEG = _NPAD // _NS
_NB = 1280
_ER = 2500
_F32 = jnp.float32
_SC_PARAMS = pltpu.CompilerParams(needs_layout_passes=False)


def _mesh():
    return plsc.VectorSubcoreMesh(core_axis_name="c", subcore_axis_name="s",
                                  num_cores=_NC, num_subcores=_NS)


def _mask_body(vh_ref, vv_ref, m_ref, oh_ref, ov_ref):
    sig = 1.0 / (1.0 + jnp.exp(-m_ref[...]))
    oh_ref[...] = vh_ref[...] * sig
    ov_ref[...] = vv_ref[...] * sig


def _tc_mask(values_hor, values_ver, mask):
    vh = values_hor.reshape(_ER, _D)
    vv = values_ver.reshape(_ER, _D)
    m = mask.reshape(_ER, _D)
    oh, ov = pl.pallas_call(
        _mask_body,
        out_shape=[jax.ShapeDtypeStruct((_ER, _D), _F32)] * 2,
    )(vh, vv, m)
    return oh.reshape(_E), ov.reshape(_E)


def _stream_chunks(base, in_hbm, bufs, sems, body):
    def start(ci, parity):
        off = base + ci * _CH
        return [pltpu.async_copy(r.at[pl.ds(off, _CH)], bufs[parity][k],
                                 sems[parity])
                for k, r in enumerate(in_hbm)]

    pend = {0: start(0, 0)}
    for ci in range(_NCHUNK):
        parity = ci % 2
        if ci + 1 < _NCHUNK:
            pend[(ci + 1) % 2] = start(ci + 1, (ci + 1) % 2)
        for cp in pend.pop(parity):
            cp.wait()
        body(ci, parity)


def _pass1_body(mv_hbm, src_hbm, dst_hbm, nidx_hbm,
                w_out,
                bufs_flat, nbuf, wacc, wrow, wseg, w_stage,
                sem_a, sem_b):
    c = lax.axis_index("c")
    s = lax.axis_index("s")
    base = (c * _NS + s) * _EPW
    bufs = [bufs_flat[0:3], bufs_flat[3:6]]

    def zero_w(i, _):
        for u in range(8):
            wacc[pl.ds(i * 128 + u * 16, 16)] = jnp.zeros((16,), _F32)
        return 0
    lax.fori_loop(0, _NPAD // 128, zero_w, 0)

    pltpu.sync_copy(nidx_hbm, nbuf)
    nv = nbuf[...]

    def chunk_body(ci, parity):
        buf_v, buf_s, buf_d = bufs[parity]

        def grp(gi, _):
            for u in range(_UNROLL):
                sl = pl.ds((gi * _UNROLL + u) * 16, 16)
                eq = buf_d[sl] == nv
                plsc.addupdate_scatter(wacc, [buf_s[sl]], buf_v[sl], mask=eq)
            return 0
        lax.fori_loop(0, _GRP // _UNROLL, grp, 0)

    _stream_chunks(base, [mv_hbm, src_hbm, dst_hbm],
                   bufs, [sem_a, sem_b], chunk_body)

    pltpu.sync_copy(wacc, w_stage.at[s])
    plsc.subcore_barrier()

    def zero_seg(i, _):
        for u in range(8):
            wseg[pl.ds(i * 128 + u * 16, 16)] = jnp.zeros((16,), _F32)
        return 0
    lax.fori_loop(0, _SEG // 128, zero_seg, 0)

    def addrow(t, _):
        pltpu.sync_copy(w_stage.at[t, pl.ds(s * _SEG, _SEG)], wrow)

        def g2(i, _):
            for u in range(8):
                sl = pl.ds(i * 128 + u * 16, 16)
                wseg[sl] = wseg[sl] + wrow[sl]
            return 0
        lax.fori_loop(0, _SEG // 128, g2, 0)
        return 0
    lax.fori_loop(0, _NS, addrow, 0)
    pltpu.sync_copy(wseg, w_out.at[c, pl.ds(s * _SEG, _SEG)])


def _sc_pass1(mv, src_v, dst_v, nidx):
    f = pl.kernel(
        _pass1_body,
        out_type=jax.ShapeDtypeStruct((_NC, _NPAD), _F32),
        mesh=_mesh(),
        compiler_params=_SC_PARAMS,
        scratch_types=[
            [pltpu.VMEM((_CH,), _F32),
             pltpu.VMEM((_CH,), jnp.int32), pltpu.VMEM((_CH,), jnp.int32),
             pltpu.VMEM((_CH,), _F32),
             pltpu.VMEM((_CH,), jnp.int32), pltpu.VMEM((_CH,), jnp.int32)],
            pltpu.VMEM((16,), jnp.int32),
            pltpu.VMEM((_NPAD,), _F32),
            pltpu.VMEM((_SEG,), _F32),
            pltpu.VMEM((_SEG,), _F32),
            pltpu.VMEM_SHARED((_NS, _NPAD), _F32),
            pltpu.SemaphoreType.DMA,
            pltpu.SemaphoreType.DMA,
        ],
    )
    return f(mv, src_v, dst_v, nidx)


def _pass2_body(mh_hbm, src_hbm, dst_hbm, w_hbm, x_hbm,
                g_out,
                bufs_flat, w0, w1, seen, rows, rows2, zrows,
                pend_d, pend_s, pend_m, pcnt, g_shared,
                sem_a, sem_b, sem_g, sem_h):
    c = lax.axis_index("c")
    s = lax.axis_index("s")
    base = (c * _NS + s) * _EPW
    bufs = [bufs_flat[0:3], bufs_flat[3:6]]

    for ci in range(_NPAD // 2048):
        off = ci * 2048
        pltpu.sync_copy(w_hbm.at[0, pl.ds(off, 2048)], w0)
        pltpu.sync_copy(w_hbm.at[1, pl.ds(off, 2048)], w1)

        def wsum(i, _):
            for u in range(8):
                sl = pl.ds(i * 128 + u * 16, 16)
                seen[pl.ds(off + i * 128 + u * 16, 16)] = w0[sl] + w1[sl]
            return 0
        lax.fori_loop(0, 16, wsum, 0)

    def zz(i, _):
        for u in range(8):
            zrows[i, pl.ds(u * 16, 16)] = jnp.zeros((16,), _F32)
        return 0
    lax.fori_loop(0, 16, zz, 0)

    def zg(k, _):
        g = k * _NS + s
        nhit = plsc.all_reduce_population_count(
            seen[pl.ds(g * 16, 16)] != 0.0)[0]

        @pl.when(nhit > 0)
        def _():
            pltpu.sync_copy(zrows, g_shared.at[pl.ds(g * 16, 16)])
        return 0
    lax.fori_loop(0, _GPT, zg, 0)
    plsc.subcore_barrier()

    zidx = jnp.zeros((16,), jnp.int32)

    def _scale_and_scatter(rbuf, j):
        dv = pend_d[pl.ds(j * 16, 16)]
        mhv = pend_m[pl.ds(j * 16, 16)]
        for r in range(16):
            sr = lax.gather(
                mhv, jnp.full((16, 1), r, jnp.int32),
                lax.GatherDimensionNumbers(
                    offset_dims=(), collapsed_slice_dims=(0,),
                    start_index_map=(0,)),
                (1,), mode=lax.GatherScatterMode.PROMISE_IN_BOUNDS)
            for cg in range(8):
                rsl = pl.ds(cg * 16, 16)
                rbuf[r, rsl] = rbuf[r, rsl] * sr
        pltpu.sync_copy(rbuf, g_shared.at[dv], add=True)

    def chunk_body(ci, parity):
        buf_v, buf_s, buf_d = bufs[parity]

        def grp(gi, cnt):
            svs = []
            for u in range(_UNROLL):
                sl = pl.ds((gi * _UNROLL + u) * 16, 16)
                svs.append(plsc.load_gather(seen, [buf_d[sl]]))
            blk = svs[0]
            for u in range(1, _UNROLL):
                blk = jnp.maximum(blk, svs[u])
            nblk = plsc.all_reduce_population_count(blk != 0.0)[0]

            @pl.when(nblk > 0)
            def _():
                c2 = cnt
                for u in range(_UNROLL):
                    sl = pl.ds((gi * _UNROLL + u) * 16, 16)
                    msk = svs[u] != 0.0
                    nhit = plsc.all_reduce_population_count(msk)[0]
                    plsc.store_compressed(pend_d.at[pl.ds(c2, 16)],
                                          buf_d[sl], mask=msk)
                    plsc.store_compressed(pend_s.at[pl.ds(c2, 16)],
                                          buf_s[sl], mask=msk)
                    plsc.store_compressed(pend_m.at[pl.ds(c2, 16)],
                                          buf_v[sl], mask=msk)
                    c2 = c2 + nhit
                pcnt[0] = c2
            return jnp.where(nblk > 0, pcnt[0], cnt)
        cnt = lax.fori_loop(0, _GRP // _UNROLL, grp, 0)
        pend_d[pl.ds(cnt, 16)] = jnp.zeros((16,), jnp.int32)
        pend_s[pl.ds(cnt, 16)] = jnp.zeros((16,), jnp.int32)
        pend_m[pl.ds(cnt, 16)] = jnp.zeros((16,), _F32)
        np_ = (cnt + 15) // 16

        @pl.when(np_ > 0)
        def _():
            pltpu.async_copy(x_hbm.at[pend_s[pl.ds(0, 16)]], rows, sem_g)

        def fl(j, _):
            @pl.when(j % 2 == 0)
            def _():
                pltpu.make_async_copy(x_hbm.at[zidx], rows, sem_g).wait()

                @pl.when(j + 1 < np_)
                def _():
                    pltpu.async_copy(
                        x_hbm.at[pend_s[pl.ds((j + 1) * 16, 16)]],
                        rows2, sem_h)
                _scale_and_scatter(rows, j)

            @pl.when(j % 2 == 1)
            def _():
                pltpu.make_async_copy(x_hbm.at[zidx], rows2, sem_h).wait()

                @pl.when(j + 1 < np_)
                def _():
                    pltpu.async_copy(
                        x_hbm.at[pend_s[pl.ds((j + 1) * 16, 16)]],
                        rows, sem_g)
                _scale_and_scatter(rows2, j)
            return 0
        lax.fori_loop(0, np_, fl, 0)

    _stream_chunks(base, [mh_hbm, src_hbm, dst_hbm],
                   bufs, [sem_a, sem_b], chunk_body)
    plsc.subcore_barrier()

    def xg(k, _):
        g = k * _NS + s
        rsl = pl.ds(g * 16, 16)
        nhit = plsc.all_reduce_population_count(seen[rsl] != 0.0)[0]

        @pl.when(nhit > 0)
        def _():
            pltpu.sync_copy(g_shared.at[rsl], g_out.at[c, rsl])
        return 0
    lax.fori_loop(0, _GPT, xg, 0)


def _sc_pass2(mh, src_h, dst_h, w_part, X):
    f = pl.kernel(
        _pass2_body,
        out_type=jax.ShapeDtypeStruct((_NC, _NPAD, _D), _F32),
        mesh=_mesh(),
        compiler_params=_SC_PARAMS,
        scratch_types=[
            [pltpu.VMEM((_CH,), _F32),
             pltpu.VMEM((_CH,), jnp.int32), pltpu.VMEM((_CH,), jnp.int32),
             pltpu.VMEM((_CH,), _F32),
             pltpu.VMEM((_CH,), jnp.int32), pltpu.VMEM((_CH,), jnp.int32)],
            pltpu.VMEM((2048,), _F32),
            pltpu.VMEM((2048,), _F32),
            pltpu.VMEM((_NPAD,), _F32),
            pltpu.VMEM((16, _D), _F32),
            pltpu.VMEM((16, _D), _F32),
            pltpu.VMEM((16, _D), _F32),
            pltpu.VMEM((_CH + 16,), jnp.int32),
            pltpu.VMEM((_CH + 16,), jnp.int32),
            pltpu.VMEM((_CH + 16,), _F32),
            pltpu.SMEM((1,), jnp.int32),
            pltpu.VMEM_SHARED((_NPAD, _D), _F32),
            pltpu.SemaphoreType.DMA,
            pltpu.SemaphoreType.DMA,
            pltpu.SemaphoreType.DMA,
            pltpu.SemaphoreType.DMA,
        ],
    )
    return f(mh, src_h, dst_h, w_part, X)


def _final_body(w_ref, g_ref, w1_ref, w2_ref, o_ref, acc_ref):
    i = pl.program_id(0)

    @pl.when(i == 0)
    def _():
        acc_ref[...] = jnp.zeros((1, _D), _F32)

    ws2 = w_ref[:, pl.ds(i * _NB, _NB)]
    wsum = ws2[0:1, :] + ws2[1:2, :]
    g0 = g_ref[0]
    g1 = g_ref[1]
    gsum = jnp.where(jnp.isfinite(g0), g0, 0.0) + \
        jnp.where(jnp.isfinite(g1), g1, 0.0)
    t = jnp.maximum(jnp.dot(gsum, w1_ref[...], precision=lax.Precision.HIGHEST,
                            preferred_element_type=_F32), 0.0)
    acc_ref[...] += jnp.dot(wsum, t, precision=lax.Precision.HIGHEST,
                            preferred_element_type=_F32)

    @pl.when(i == (_NPAD // _NB) - 1)
    def _():
        z = jnp.dot(acc_ref[...], w2_ref[...],
                    precision=lax.Precision.HIGHEST,
                    preferred_element_type=_F32)
        z = z - jnp.max(z, axis=-1, keepdims=True)
        e = jnp.exp(z)
        o_ref[...] = e / jnp.sum(e, axis=-1, keepdims=True)


def _tc_final(w_part, g_part, W1, W2):
    nsteps = _NPAD // _NB
    return pl.pallas_call(
        _final_body,
        grid=(nsteps,),
        in_specs=[
            pl.BlockSpec((_NC, _NPAD), lambda i: (0, 0)),
            pl.BlockSpec((_NC, _NB, _D), lambda i: (0, i, 0)),
            pl.BlockSpec((_D, _D), lambda i: (0, 0)),
            pl.BlockSpec((_D, _C), lambda i: (0, 0)),
        ],
        out_specs=pl.BlockSpec((1, _C), lambda i: (0, 0)),
        out_shape=jax.ShapeDtypeStruct((1, _C), _F32),
        scratch_shapes=[pltpu.VMEM((1, _D), _F32)],
    )(w_part, g_part, W1, W2)


def kernel(values_hor, values_ver, mask, X, W1, W2,
           edge_index_hor, edge_index_ver, node_idx):
    src_h = edge_index_hor[0]
    dst_h = edge_index_hor[1]
    src_v = edge_index_ver[0]
    dst_v = edge_index_ver[1]
    nidx = jnp.full((16,), node_idx, jnp.int32)

    masked_hor, masked_ver = _tc_mask(values_hor, values_ver, mask)
    w_part = _sc_pass1(masked_ver, src_v, dst_v, nidx)
    g_part = _sc_pass2(masked_hor, src_h, dst_h, w_part, X)
    res = _tc_final(w_part, g_part, W1, W2).reshape(_C)
    return (res, masked_hor, masked_ver)


# speedup vs baseline: 1.2896x; 1.2896x over previous
import jax
import jax.numpy as jnp
from jax import lax
from jax.experimental import pallas as pl
from jax.experimental.pallas import tpu as pltpu
from jax.experimental.pallas import tpu_sc as plsc

_N = 10000
_E = 320000
_D = 128
_C = 8
_NC = 2
_NS = 16
_NW = _NC * _NS
_EPW = _E // _NW
_CH = 2000
_NCHUNK = _EPW // _CH
_GRP = _CH // 16
_UNROLL = 5
_NPAD = 10240
_NGRP = _NPAD // 16
_GPT = _NGRP // _NS
_SEG = _NPAD // _NS
_NB = 1280
_ER = 2500
_F32 = jnp.float32
_SC_PARAMS = pltpu.CompilerParams(needs_layout_passes=False)


def _mesh():
    return plsc.VectorSubcoreMesh(core_axis_name="c", subcore_axis_name="s",
                                  num_cores=_NC, num_subcores=_NS)


def _sigmoid_sc(m):
    y = m * (-1.4426950408889634)
    y = jnp.minimum(jnp.maximum(y, -126.0), 126.0)
    fy = (y + 12582912.0) - 12582912.0
    u = (y - fy) * 0.6931471805599453
    p = 1.0 + u * (1.0 + u * (0.5 + u * (
        0.16666666666666666 + u * (0.041666666666666664 + u * (
            0.008333333333333333 + u * (
                0.001388888888888889 + u * 0.0001984126984126984))))))
    n = fy.astype(jnp.int32)
    sc = plsc.bitcast((n + 127) << 23, _F32)
    return 1.0 / (1.0 + p * sc)


def _mask_body(vh_ref, vv_ref, m_ref, oh_ref, ov_ref):
    sig = 1.0 / (1.0 + jnp.exp(-m_ref[...]))
    oh_ref[...] = vh_ref[...] * sig
    ov_ref[...] = vv_ref[...] * sig


def _tc_mask(values_hor, values_ver, mask):
    vh = values_hor.reshape(_ER, _D)
    vv = values_ver.reshape(_ER, _D)
    m = mask.reshape(_ER, _D)
    oh, ov = pl.pallas_call(
        _mask_body,
        out_shape=[jax.ShapeDtypeStruct((_ER, _D), _F32)] * 2,
    )(vh, vv, m)
    return oh.reshape(_E), ov.reshape(_E)


def _stream_chunks(base, in_hbm, bufs, sems, body):
    def start(ci, parity):
        off = base + ci * _CH
        return [pltpu.async_copy(r.at[pl.ds(off, _CH)], bufs[parity][k],
                                 sems[parity])
                for k, r in enumerate(in_hbm)]

    pend = {0: start(0, 0)}
    for ci in range(_NCHUNK):
        parity = ci % 2
        if ci + 1 < _NCHUNK:
            pend[(ci + 1) % 2] = start(ci + 1, (ci + 1) % 2)
        for cp in pend.pop(parity):
            cp.wait()
        body(ci, parity)


def _pass1_body(vv_hbm, m_hbm, src_hbm, dst_hbm, nidx_hbm,
                w_out,
                bufs_flat, nbuf, wacc, wrow, wseg, w_stage,
                sem_a, sem_b):
    c = lax.axis_index("c")
    s = lax.axis_index("s")
    base = (c * _NS + s) * _EPW
    bufs = [bufs_flat[0:4], bufs_flat[4:8]]

    def zero_w(i, _):
        for u in range(8):
            wacc[pl.ds(i * 128 + u * 16, 16)] = jnp.zeros((16,), _F32)
        return 0
    lax.fori_loop(0, _NPAD // 128, zero_w, 0)

    pltpu.sync_copy(nidx_hbm, nbuf)
    nv = nbuf[...]

    def chunk_body(ci, parity):
        buf_v, buf_m, buf_s, buf_d = bufs[parity]

        def grp(gi, _):
            eqs = []
            for u in range(_UNROLL):
                sl = pl.ds((gi * _UNROLL + u) * 16, 16)
                eqs.append(buf_d[sl] == nv)
            blk = eqs[0]
            for u in range(1, _UNROLL):
                blk = blk | eqs[u]
            nblk = plsc.all_reduce_population_count(blk)[0]

            @pl.when(nblk > 0)
            def _():
                for u in range(_UNROLL):
                    sl = pl.ds((gi * _UNROLL + u) * 16, 16)
                    mv = buf_v[sl] * _sigmoid_sc(buf_m[sl])
                    plsc.addupdate_scatter(wacc, [buf_s[sl]], mv,
                                           mask=eqs[u])
            return 0
        lax.fori_loop(0, _GRP // _UNROLL, grp, 0)

    _stream_chunks(base, [vv_hbm, m_hbm, src_hbm, dst_hbm],
                   bufs, [sem_a, sem_b], chunk_body)

    pltpu.sync_copy(wacc, w_stage.at[s])
    plsc.subcore_barrier()

    def zero_seg(i, _):
        for u in range(8):
            wseg[pl.ds(i * 128 + u * 16, 16)] = jnp.zeros((16,), _F32)
        return 0
    lax.fori_loop(0, _SEG // 128, zero_seg, 0)

    def addrow(t, _):
        pltpu.sync_copy(w_stage.at[t, pl.ds(s * _SEG, _SEG)], wrow)

        def g2(i, _):
            for u in range(8):
                sl = pl.ds(i * 128 + u * 16, 16)
                wseg[sl] = wseg[sl] + wrow[sl]
            return 0
        lax.fori_loop(0, _SEG // 128, g2, 0)
        return 0
    lax.fori_loop(0, _NS, addrow, 0)
    pltpu.sync_copy(wseg, w_out.at[c, pl.ds(s * _SEG, _SEG)])


def _sc_pass1(values_ver, mask, src_v, dst_v, nidx):
    f = pl.kernel(
        _pass1_body,
        out_type=jax.ShapeDtypeStruct((_NC, _NPAD), _F32),
        mesh=_mesh(),
        compiler_params=_SC_PARAMS,
        scratch_types=[
            [pltpu.VMEM((_CH,), _F32), pltpu.VMEM((_CH,), _F32),
             pltpu.VMEM((_CH,), jnp.int32), pltpu.VMEM((_CH,), jnp.int32),
             pltpu.VMEM((_CH,), _F32), pltpu.VMEM((_CH,), _F32),
             pltpu.VMEM((_CH,), jnp.int32), pltpu.VMEM((_CH,), jnp.int32)],
            pltpu.VMEM((16,), jnp.int32),
            pltpu.VMEM((_NPAD,), _F32),
            pltpu.VMEM((_SEG,), _F32),
            pltpu.VMEM((_SEG,), _F32),
            pltpu.VMEM_SHARED((_NS, _NPAD), _F32),
            pltpu.SemaphoreType.DMA,
            pltpu.SemaphoreType.DMA,
        ],
    )
    return f(values_ver, mask, src_v, dst_v, nidx)


def _pass2_body(vh_hbm, m_hbm, src_hbm, dst_hbm, w_hbm, x_hbm,
                g_out,
                bufs_flat, w0, w1, seen, rows, rows2, zrows,
                pend_d, pend_s, pend_m, pcnt, g_shared,
                sem_a, sem_b, sem_g, sem_h):
    c = lax.axis_index("c")
    s = lax.axis_index("s")
    base = (c * _NS + s) * _EPW
    bufs = [bufs_flat[0:4], bufs_flat[4:8]]

    for ci in range(_NPAD // 2048):
        off = ci * 2048
        pltpu.sync_copy(w_hbm.at[0, pl.ds(off, 2048)], w0)
        pltpu.sync_copy(w_hbm.at[1, pl.ds(off, 2048)], w1)

        def wsum(i, _):
            for u in range(8):
                sl = pl.ds(i * 128 + u * 16, 16)
                seen[pl.ds(off + i * 128 + u * 16, 16)] = w0[sl] + w1[sl]
            return 0
        lax.fori_loop(0, 16, wsum, 0)

    def zz(i, _):
        for u in range(8):
            zrows[i, pl.ds(u * 16, 16)] = jnp.zeros((16,), _F32)
        return 0
    lax.fori_loop(0, 16, zz, 0)

    def zg(k, _):
        g = k * _NS + s
        nhit = plsc.all_reduce_population_count(
            seen[pl.ds(g * 16, 16)] != 0.0)[0]

        @pl.when(nhit > 0)
        def _():
            pltpu.sync_copy(zrows, g_shared.at[pl.ds(g * 16, 16)])
        return 0
    lax.fori_loop(0, _GPT, zg, 0)
    plsc.subcore_barrier()

    zidx = jnp.zeros((16,), jnp.int32)

    def _scale_and_scatter(rbuf, j):
        dv = pend_d[pl.ds(j * 16, 16)]
        mhv = pend_m[pl.ds(j * 16, 16)]
        for r in range(16):
            sr = lax.gather(
                mhv, jnp.full((16, 1), r, jnp.int32),
                lax.GatherDimensionNumbers(
                    offset_dims=(), collapsed_slice_dims=(0,),
                    start_index_map=(0,)),
                (1,), mode=lax.GatherScatterMode.PROMISE_IN_BOUNDS)
            for cg in range(8):
                rsl = pl.ds(cg * 16, 16)
                rbuf[r, rsl] = rbuf[r, rsl] * sr
        pltpu.sync_copy(rbuf, g_shared.at[dv], add=True)

    def chunk_body(ci, parity):
        buf_v, buf_m, buf_s, buf_d = bufs[parity]

        def grp(gi, cnt):
            svs = []
            for u in range(_UNROLL):
                sl = pl.ds((gi * _UNROLL + u) * 16, 16)
                svs.append(plsc.load_gather(seen, [buf_d[sl]]))
            blk = svs[0]
            for u in range(1, _UNROLL):
                blk = jnp.maximum(blk, svs[u])
            nblk = plsc.all_reduce_population_count(blk != 0.0)[0]

            @pl.when(nblk > 0)
            def _():
                c2 = cnt
                for u in range(_UNROLL):
                    sl = pl.ds((gi * _UNROLL + u) * 16, 16)
                    msk = svs[u] != 0.0
                    nhit = plsc.all_reduce_population_count(msk)[0]

                    @pl.when(nhit > 0)
                    def _(u=u, sl=sl, msk=msk, c2=c2):
                        psl = pl.ds(c2 * 16, 16)
                        pend_d[psl] = buf_d[sl]
                        pend_s[psl] = buf_s[sl]
                        mh = buf_v[sl] * _sigmoid_sc(buf_m[sl])
                        pend_m[psl] = jnp.where(msk, mh, 0.0)
                    c2 = c2 + jnp.where(nhit > 0, 1, 0)
                pcnt[0] = c2
            return jnp.where(nblk > 0, pcnt[0], cnt)
        np_ = lax.fori_loop(0, _GRP // _UNROLL, grp, 0)

        @pl.when(np_ > 0)
        def _():
            pltpu.async_copy(x_hbm.at[pend_s[pl.ds(0, 16)]], rows, sem_g)

        def fl(j, _):
            @pl.when(j % 2 == 0)
            def _():
                pltpu.make_async_copy(x_hbm.at[zidx], rows, sem_g).wait()

                @pl.when(j + 1 < np_)
                def _():
                    pltpu.async_copy(
                        x_hbm.at[pend_s[pl.ds((j + 1) * 16, 16)]],
                        rows2, sem_h)
                _scale_and_scatter(rows, j)

            @pl.when(j % 2 == 1)
            def _():
                pltpu.make_async_copy(x_hbm.at[zidx], rows2, sem_h).wait()

                @pl.when(j + 1 < np_)
                def _():
                    pltpu.async_copy(
                        x_hbm.at[pend_s[pl.ds((j + 1) * 16, 16)]],
                        rows, sem_g)
                _scale_and_scatter(rows2, j)
            return 0
        lax.fori_loop(0, np_, fl, 0)

    _stream_chunks(base, [vh_hbm, m_hbm, src_hbm, dst_hbm],
                   bufs, [sem_a, sem_b], chunk_body)
    plsc.subcore_barrier()

    def xg(k, _):
        g = k * _NS + s
        rsl = pl.ds(g * 16, 16)
        nhit = plsc.all_reduce_population_count(seen[rsl] != 0.0)[0]

        @pl.when(nhit > 0)
        def _():
            pltpu.sync_copy(g_shared.at[rsl], g_out.at[c, rsl])
        return 0
    lax.fori_loop(0, _GPT, xg, 0)


def _sc_pass2(values_hor, mask, src_h, dst_h, w_part, X):
    f = pl.kernel(
        _pass2_body,
        out_type=jax.ShapeDtypeStruct((_NC, _NPAD, _D), _F32),
        mesh=_mesh(),
        compiler_params=_SC_PARAMS,
        scratch_types=[
            [pltpu.VMEM((_CH,), _F32), pltpu.VMEM((_CH,), _F32),
             pltpu.VMEM((_CH,), jnp.int32), pltpu.VMEM((_CH,), jnp.int32),
             pltpu.VMEM((_CH,), _F32), pltpu.VMEM((_CH,), _F32),
             pltpu.VMEM((_CH,), jnp.int32), pltpu.VMEM((_CH,), jnp.int32)],
            pltpu.VMEM((2048,), _F32),
            pltpu.VMEM((2048,), _F32),
            pltpu.VMEM((_NPAD,), _F32),
            pltpu.VMEM((16, _D), _F32),
            pltpu.VMEM((16, _D), _F32),
            pltpu.VMEM((16, _D), _F32),
            pltpu.VMEM((_CH + 16,), jnp.int32),
            pltpu.VMEM((_CH + 16,), jnp.int32),
            pltpu.VMEM((_CH + 16,), _F32),
            pltpu.SMEM((1,), jnp.int32),
            pltpu.VMEM_SHARED((_NPAD, _D), _F32),
            pltpu.SemaphoreType.DMA,
            pltpu.SemaphoreType.DMA,
            pltpu.SemaphoreType.DMA,
            pltpu.SemaphoreType.DMA,
        ],
    )
    return f(values_hor, mask, src_h, dst_h, w_part, X)


def _final_body(w_ref, g_ref, w1_ref, w2_ref, o_ref, acc_ref):
    i = pl.program_id(0)

    @pl.when(i == 0)
    def _():
        acc_ref[...] = jnp.zeros((1, _D), _F32)

    ws2 = w_ref[:, pl.ds(i * _NB, _NB)]
    wsum = ws2[0:1, :] + ws2[1:2, :]
    g0 = g_ref[0]
    g1 = g_ref[1]
    gsum = jnp.where(jnp.isfinite(g0), g0, 0.0) + \
        jnp.where(jnp.isfinite(g1), g1, 0.0)
    t = jnp.maximum(jnp.dot(gsum, w1_ref[...], precision=lax.Precision.HIGHEST,
                            preferred_element_type=_F32), 0.0)
    acc_ref[...] += jnp.dot(wsum, t, precision=lax.Precision.HIGHEST,
                            preferred_element_type=_F32)

    @pl.when(i == (_NPAD // _NB) - 1)
    def _():
        z = jnp.dot(acc_ref[...], w2_ref[...],
                    precision=lax.Precision.HIGHEST,
                    preferred_element_type=_F32)
        z = z - jnp.max(z, axis=-1, keepdims=True)
        e = jnp.exp(z)
        o_ref[...] = e / jnp.sum(e, axis=-1, keepdims=True)


def _tc_final(w_part, g_part, W1, W2):
    nsteps = _NPAD // _NB
    return pl.pallas_call(
        _final_body,
        grid=(nsteps,),
        in_specs=[
            pl.BlockSpec((_NC, _NPAD), lambda i: (0, 0)),
            pl.BlockSpec((_NC, _NB, _D), lambda i: (0, i, 0)),
            pl.BlockSpec((_D, _D), lambda i: (0, 0)),
            pl.BlockSpec((_D, _C), lambda i: (0, 0)),
        ],
        out_specs=pl.BlockSpec((1, _C), lambda i: (0, 0)),
        out_shape=jax.ShapeDtypeStruct((1, _C), _F32),
        scratch_shapes=[pltpu.VMEM((1, _D), _F32)],
    )(w_part, g_part, W1, W2)


def kernel(values_hor, values_ver, mask, X, W1, W2,
           edge_index_hor, edge_index_ver, node_idx):
    src_h = edge_index_hor[0]
    dst_h = edge_index_hor[1]
    src_v = edge_index_ver[0]
    dst_v = edge_index_ver[1]
    nidx = jnp.full((16,), node_idx, jnp.int32)

    masked_hor, masked_ver = _tc_mask(values_hor, values_ver, mask)
    w_part = _sc_pass1(values_ver, mask, src_v, dst_v, nidx)
    g_part = _sc_pass2(values_hor, mask, src_h, dst_h, w_part, X)
    res = _tc_final(w_part, g_part, W1, W2).reshape(_C)
    return (res, masked_hor, masked_ver)


# speedup vs baseline: 1.3400x; 1.0390x over previous
import jax
import jax.numpy as jnp
from jax import lax
from jax.experimental import pallas as pl
from jax.experimental.pallas import tpu as pltpu
from jax.experimental.pallas import tpu_sc as plsc

_N = 10000
_E = 320000
_D = 128
_C = 8
_NC = 2
_NS = 16
_NW = _NC * _NS
_EPW = _E // _NW
_CH = 2000
_NCHUNK = _EPW // _CH
_GRP = _CH // 16
_UNROLL = 5
_NPAD = 10240
_NGRP = _NPAD // 16
_GPT = _NGRP // _NS
_SEG = _NPAD // _NS
_NB = 1280
_ER = 2500
_F32 = jnp.float32
_SC_PARAMS = pltpu.CompilerParams(needs_layout_passes=False)


def _mesh():
    return plsc.VectorSubcoreMesh(core_axis_name="c", subcore_axis_name="s",
                                  num_cores=_NC, num_subcores=_NS)


def _mask_body(vh_ref, vv_ref, m_ref, oh_ref, ov_ref):
    sig = 1.0 / (1.0 + jnp.exp(-m_ref[...]))
    oh_ref[...] = vh_ref[...] * sig
    ov_ref[...] = vv_ref[...] * sig


def _tc_mask(values_hor, values_ver, mask):
    vh = values_hor.reshape(_ER, _D)
    vv = values_ver.reshape(_ER, _D)
    m = mask.reshape(_ER, _D)
    oh, ov = pl.pallas_call(
        _mask_body,
        out_shape=[jax.ShapeDtypeStruct((_ER, _D), _F32)] * 2,
    )(vh, vv, m)
    return oh.reshape(_E), ov.reshape(_E)


def _stream_chunks(base, in_hbm, bufs, sems, body):
    def start(ci, parity):
        off = base + ci * _CH
        return [pltpu.async_copy(r.at[pl.ds(off, _CH)], bufs[parity][k],
                                 sems[parity])
                for k, r in enumerate(in_hbm)]

    pend = {0: start(0, 0)}
    for ci in range(_NCHUNK):
        parity = ci % 2
        if ci + 1 < _NCHUNK:
            pend[(ci + 1) % 2] = start(ci + 1, (ci + 1) % 2)
        for cp in pend.pop(parity):
            cp.wait()
        body(ci, parity)


def _pass1_body(mv_hbm, src_hbm, dst_hbm, nidx_hbm,
                w_out,
                bufs_flat, nbuf, wacc, wrow, wseg, w_stage,
                sem_a, sem_b):
    c = lax.axis_index("c")
    s = lax.axis_index("s")
    base = (c * _NS + s) * _EPW
    bufs = [bufs_flat[0:3], bufs_flat[3:6]]

    def zero_w(i, _):
        for u in range(8):
            wacc[pl.ds(i * 128 + u * 16, 16)] = jnp.zeros((16,), _F32)
        return 0
    lax.fori_loop(0, _NPAD // 128, zero_w, 0)

    pltpu.sync_copy(nidx_hbm, nbuf)
    nv = nbuf[...]

    def chunk_body(ci, parity):
        buf_v, buf_s, buf_d = bufs[parity]

        def grp(gi, _):
            for u in range(_UNROLL):
                sl = pl.ds((gi * _UNROLL + u) * 16, 16)
                eq = buf_d[sl] == nv
                plsc.addupdate_scatter(wacc, [buf_s[sl]], buf_v[sl], mask=eq)
            return 0
        lax.fori_loop(0, _GRP // _UNROLL, grp, 0)

    _stream_chunks(base, [mv_hbm, src_hbm, dst_hbm],
                   bufs, [sem_a, sem_b], chunk_body)

    pltpu.sync_copy(wacc, w_stage.at[s])
    plsc.subcore_barrier()

    def zero_seg(i, _):
        for u in range(8):
            wseg[pl.ds(i * 128 + u * 16, 16)] = jnp.zeros((16,), _F32)
        return 0
    lax.fori_loop(0, _SEG // 128, zero_seg, 0)

    def addrow(t, _):
        pltpu.sync_copy(w_stage.at[t, pl.ds(s * _SEG, _SEG)], wrow)

        def g2(i, _):
            for u in range(8):
                sl = pl.ds(i * 128 + u * 16, 16)
                wseg[sl] = wseg[sl] + wrow[sl]
            return 0
        lax.fori_loop(0, _SEG // 128, g2, 0)
        return 0
    lax.fori_loop(0, _NS, addrow, 0)
    pltpu.sync_copy(wseg, w_out.at[c, pl.ds(s * _SEG, _SEG)])


def _sc_pass1(mv, src_v, dst_v, nidx):
    f = pl.kernel(
        _pass1_body,
        out_type=jax.ShapeDtypeStruct((_NC, _NPAD), _F32),
        mesh=_mesh(),
        compiler_params=_SC_PARAMS,
        scratch_types=[
            [pltpu.VMEM((_CH,), _F32),
             pltpu.VMEM((_CH,), jnp.int32), pltpu.VMEM((_CH,), jnp.int32),
             pltpu.VMEM((_CH,), _F32),
             pltpu.VMEM((_CH,), jnp.int32), pltpu.VMEM((_CH,), jnp.int32)],
            pltpu.VMEM((16,), jnp.int32),
            pltpu.VMEM((_NPAD,), _F32),
            pltpu.VMEM((_SEG,), _F32),
            pltpu.VMEM((_SEG,), _F32),
            pltpu.VMEM_SHARED((_NS, _NPAD), _F32),
            pltpu.SemaphoreType.DMA,
            pltpu.SemaphoreType.DMA,
        ],
    )
    return f(mv, src_v, dst_v, nidx)


def _pass2_body(mh_hbm, src_hbm, dst_hbm, w_hbm, x_hbm,
                g_out,
                bufs_flat, w0, w1, seen, rows, rows2, zrows,
                pend_d, pend_s, pend_m, pcnt, g_shared,
                sem_a, sem_b, sem_g, sem_h):
    c = lax.axis_index("c")
    s = lax.axis_index("s")
    base = (c * _NS + s) * _EPW
    bufs = [bufs_flat[0:3], bufs_flat[3:6]]

    for ci in range(_NPAD // 2048):
        off = ci * 2048
        pltpu.sync_copy(w_hbm.at[0, pl.ds(off, 2048)], w0)
        pltpu.sync_copy(w_hbm.at[1, pl.ds(off, 2048)], w1)

        def wsum(i, _):
            for u in range(8):
                sl = pl.ds(i * 128 + u * 16, 16)
                seen[pl.ds(off + i * 128 + u * 16, 16)] = w0[sl] + w1[sl]
            return 0
        lax.fori_loop(0, 16, wsum, 0)

    def zz(i, _):
        for u in range(8):
            zrows[i, pl.ds(u * 16, 16)] = jnp.zeros((16,), _F32)
        return 0
    lax.fori_loop(0, 16, zz, 0)

    def zg(k, _):
        g = k * _NS + s
        nhit = plsc.all_reduce_population_count(
            seen[pl.ds(g * 16, 16)] != 0.0)[0]

        @pl.when(nhit > 0)
        def _():
            pltpu.sync_copy(zrows, g_shared.at[pl.ds(g * 16, 16)])
        return 0
    lax.fori_loop(0, _GPT, zg, 0)
    plsc.subcore_barrier()

    zidx = jnp.zeros((16,), jnp.int32)

    def _scale_and_scatter(rbuf, j):
        dv = pend_d[pl.ds(j * 16, 16)]
        mhv = pend_m[pl.ds(j * 16, 16)]
        for r in range(16):
            sr = lax.gather(
                mhv, jnp.full((16, 1), r, jnp.int32),
                lax.GatherDimensionNumbers(
                    offset_dims=(), collapsed_slice_dims=(0,),
                    start_index_map=(0,)),
                (1,), mode=lax.GatherScatterMode.PROMISE_IN_BOUNDS)
            for cg in range(8):
                rsl = pl.ds(cg * 16, 16)
                rbuf[r, rsl] = rbuf[r, rsl] * sr
        pltpu.sync_copy(rbuf, g_shared.at[dv], add=True)

    def chunk_body(ci, parity):
        buf_v, buf_s, buf_d = bufs[parity]

        def grp(gi, cnt):
            svs = []
            for u in range(_UNROLL):
                sl = pl.ds((gi * _UNROLL + u) * 16, 16)
                svs.append(plsc.load_gather(seen, [buf_d[sl]]))
            blk = svs[0]
            for u in range(1, _UNROLL):
                blk = jnp.maximum(blk, svs[u])
            nblk = plsc.all_reduce_population_count(blk != 0.0)[0]

            @pl.when(nblk > 0)
            def _():
                c2 = cnt
                for u in range(_UNROLL):
                    sl = pl.ds((gi * _UNROLL + u) * 16, 16)
                    msk = svs[u] != 0.0
                    nhit = plsc.all_reduce_population_count(msk)[0]

                    @pl.when(nhit > 0)
                    def _(u=u, sl=sl, msk=msk, c2=c2):
                        psl = pl.ds(c2 * 16, 16)
                        pend_d[psl] = buf_d[sl]
                        pend_s[psl] = buf_s[sl]
                        pend_m[psl] = jnp.where(msk, buf_v[sl], 0.0)
                    c2 = c2 + jnp.where(nhit > 0, 1, 0)
                pcnt[0] = c2
            return jnp.where(nblk > 0, pcnt[0], cnt)
        np_ = lax.fori_loop(0, _GRP // _UNROLL, grp, 0)

        @pl.when(np_ > 0)
        def _():
            pltpu.async_copy(x_hbm.at[pend_s[pl.ds(0, 16)]], rows, sem_g)

        def fl(j, _):
            @pl.when(j % 2 == 0)
            def _():
                pltpu.make_async_copy(x_hbm.at[zidx], rows, sem_g).wait()

                @pl.when(j + 1 < np_)
                def _():
                    pltpu.async_copy(
                        x_hbm.at[pend_s[pl.ds((j + 1) * 16, 16)]],
                        rows2, sem_h)
                _scale_and_scatter(rows, j)

            @pl.when(j % 2 == 1)
            def _():
                pltpu.make_async_copy(x_hbm.at[zidx], rows2, sem_h).wait()

                @pl.when(j + 1 < np_)
                def _():
                    pltpu.async_copy(
                        x_hbm.at[pend_s[pl.ds((j + 1) * 16, 16)]],
                        rows, sem_g)
                _scale_and_scatter(rows2, j)
            return 0
        lax.fori_loop(0, np_, fl, 0)

    _stream_chunks(base, [mh_hbm, src_hbm, dst_hbm],
                   bufs, [sem_a, sem_b], chunk_body)
    plsc.subcore_barrier()

    def xg(k, _):
        g = k * _NS + s
        rsl = pl.ds(g * 16, 16)
        nhit = plsc.all_reduce_population_count(seen[rsl] != 0.0)[0]

        @pl.when(nhit > 0)
        def _():
            pltpu.sync_copy(g_shared.at[rsl], g_out.at[c, rsl])
        return 0
    lax.fori_loop(0, _GPT, xg, 0)


def _sc_pass2(mh, src_h, dst_h, w_part, X):
    f = pl.kernel(
        _pass2_body,
        out_type=jax.ShapeDtypeStruct((_NC, _NPAD, _D), _F32),
        mesh=_mesh(),
        compiler_params=_SC_PARAMS,
        scratch_types=[
            [pltpu.VMEM((_CH,), _F32),
             pltpu.VMEM((_CH,), jnp.int32), pltpu.VMEM((_CH,), jnp.int32),
             pltpu.VMEM((_CH,), _F32),
             pltpu.VMEM((_CH,), jnp.int32), pltpu.VMEM((_CH,), jnp.int32)],
            pltpu.VMEM((2048,), _F32),
            pltpu.VMEM((2048,), _F32),
            pltpu.VMEM((_NPAD,), _F32),
            pltpu.VMEM((16, _D), _F32),
            pltpu.VMEM((16, _D), _F32),
            pltpu.VMEM((16, _D), _F32),
            pltpu.VMEM((_CH + 16,), jnp.int32),
            pltpu.VMEM((_CH + 16,), jnp.int32),
            pltpu.VMEM((_CH + 16,), _F32),
            pltpu.SMEM((1,), jnp.int32),
            pltpu.VMEM_SHARED((_NPAD, _D), _F32),
            pltpu.SemaphoreType.DMA,
            pltpu.SemaphoreType.DMA,
            pltpu.SemaphoreType.DMA,
            pltpu.SemaphoreType.DMA,
        ],
    )
    return f(mh, src_h, dst_h, w_part, X)


def _final_body(w_ref, g_ref, w1_ref, w2_ref, o_ref, acc_ref):
    i = pl.program_id(0)

    @pl.when(i == 0)
    def _():
        acc_ref[...] = jnp.zeros((1, _D), _F32)

    ws2 = w_ref[:, pl.ds(i * _NB, _NB)]
    wsum = ws2[0:1, :] + ws2[1:2, :]
    g0 = g_ref[0]
    g1 = g_ref[1]
    gsum = jnp.where(jnp.isfinite(g0), g0, 0.0) + \
        jnp.where(jnp.isfinite(g1), g1, 0.0)
    t = jnp.maximum(jnp.dot(gsum, w1_ref[...], precision=lax.Precision.HIGHEST,
                            preferred_element_type=_F32), 0.0)
    acc_ref[...] += jnp.dot(wsum, t, precision=lax.Precision.HIGHEST,
                            preferred_element_type=_F32)

    @pl.when(i == (_NPAD // _NB) - 1)
    def _():
        z = jnp.dot(acc_ref[...], w2_ref[...],
                    precision=lax.Precision.HIGHEST,
                    preferred_element_type=_F32)
        z = z - jnp.max(z, axis=-1, keepdims=True)
        e = jnp.exp(z)
        o_ref[...] = e / jnp.sum(e, axis=-1, keepdims=True)


def _tc_final(w_part, g_part, W1, W2):
    nsteps = _NPAD // _NB
    return pl.pallas_call(
        _final_body,
        grid=(nsteps,),
        in_specs=[
            pl.BlockSpec((_NC, _NPAD), lambda i: (0, 0)),
            pl.BlockSpec((_NC, _NB, _D), lambda i: (0, i, 0)),
            pl.BlockSpec((_D, _D), lambda i: (0, 0)),
            pl.BlockSpec((_D, _C), lambda i: (0, 0)),
        ],
        out_specs=pl.BlockSpec((1, _C), lambda i: (0, 0)),
        out_shape=jax.ShapeDtypeStruct((1, _C), _F32),
        scratch_shapes=[pltpu.VMEM((1, _D), _F32)],
    )(w_part, g_part, W1, W2)


def kernel(values_hor, values_ver, mask, X, W1, W2,
           edge_index_hor, edge_index_ver, node_idx):
    src_h = edge_index_hor[0]
    dst_h = edge_index_hor[1]
    src_v = edge_index_ver[0]
    dst_v = edge_index_ver[1]
    nidx = jnp.full((16,), node_idx, jnp.int32)

    masked_hor, masked_ver = _tc_mask(values_hor, values_ver, mask)
    w_part = _sc_pass1(masked_ver, src_v, dst_v, nidx)
    g_part = _sc_pass2(masked_hor, src_h, dst_h, w_part, X)
    res = _tc_final(w_part, g_part, W1, W2).reshape(_C)
    return (res, masked_hor, masked_ver)
